# Initial kernel scaffold; baseline (speedup 1.0000x reference)
import jax
import jax.numpy as jnp
from jax.experimental import pallas as pl


def kernel(x, edge_index, W1, b1, W2, b2, Wm1, bm1, Wm2, bm2, gcn_weight, mlp_weight):
    raise NotImplementedError("write your pallas kernel here")


# speedup vs baseline: 17.5765x; 17.5765x over previous
import functools

import jax
import jax.numpy as jnp
from jax import lax
from jax.experimental import pallas as pl
from jax.experimental.pallas import tpu as pltpu
from jax.experimental.pallas import tpu_sc as plsc

_N = 10000
_E = 320000
_D = 128

_NC = 2
_NS = 16
_NW = _NC * _NS
_EPW = _E // _NW
_K = 80
_CH = _EPW // _K
_ZT = 10
_RPT = _N // _ZT

def _sc_degree_body(dsts_hbm, ones_hbm, zeros_hbm, out_hbm,
                    dst_v, ones_v, acc_sh):
    cid = lax.axis_index("c")
    sid = lax.axis_index("s")
    wid = cid * _NS + sid
    pltpu.sync_copy(dsts_hbm.at[wid], dst_v)
    pltpu.sync_copy(ones_hbm, ones_v)

    @pl.when(sid < _ZT)
    def _():
        pltpu.sync_copy(zeros_hbm.at[pl.ds(sid * _RPT, _RPT)],
                        acc_sh.at[pl.ds(sid * _RPT, _RPT)])

    plsc.subcore_barrier()

    def body(j, carry):
        pltpu.sync_copy(ones_v, acc_sh.at[dst_v.at[j]], add=True)
        return carry

    lax.fori_loop(0, _CH, body, 0)
    plsc.subcore_barrier()

    @pl.when(sid < _ZT)
    def _():
        pltpu.sync_copy(acc_sh.at[pl.ds(sid * _RPT, _RPT)],
                        out_hbm.at[pl.ds(cid * _N + sid * _RPT, _RPT)])


def _sc_edge_sum_body(y_hbm, srcs_hbm, dsts_hbm, zeros_hbm, out_hbm,
                      src_v, dst_v, rows_v, acc_sh, sem):
    cid = lax.axis_index("c")
    sid = lax.axis_index("s")
    wid = cid * _NS + sid
    pltpu.sync_copy(srcs_hbm.at[wid], src_v)
    pltpu.sync_copy(dsts_hbm.at[wid], dst_v)
    @pl.when(sid < _ZT)
    def _():
        pltpu.sync_copy(zeros_hbm.at[pl.ds(sid * _RPT, _RPT)],
                        acc_sh.at[pl.ds(sid * _RPT, _RPT)])

    plsc.subcore_barrier()

    def body(j, carry):
        pltpu.async_copy(y_hbm.at[src_v.at[j]], rows_v, sem).wait()
        pltpu.sync_copy(rows_v, acc_sh.at[dst_v.at[j]], add=True)
        return carry

    lax.fori_loop(0, _CH, body, 0)
    plsc.subcore_barrier()

    @pl.when(sid < _ZT)
    def _():
        pltpu.sync_copy(acc_sh.at[pl.ds(sid * _RPT, _RPT)],
                        out_hbm.at[pl.ds(cid * _N + sid * _RPT, _RPT)])


@functools.lru_cache(maxsize=None)
def _sc_kernels():
    mesh = plsc.VectorSubcoreMesh(core_axis_name="c", subcore_axis_name="s",
                                  num_cores=_NC, num_subcores=_NS)
    sc_degree = pl.kernel(
        _sc_degree_body,
        out_type=jax.ShapeDtypeStruct((_NC * _N, _D), jnp.float32),
        mesh=mesh,
        scratch_types=[
            pltpu.VMEM((_CH, _K), jnp.int32),
            pltpu.VMEM((_K, _D), jnp.float32),
            pltpu.VMEM_SHARED((_N, _D), jnp.float32),
        ],
    )
    sc_edge_sum = pl.kernel(
        _sc_edge_sum_body,
        out_type=jax.ShapeDtypeStruct((_NC * _N, _D), jnp.float32),
        mesh=mesh,
        scratch_types=[
            pltpu.VMEM((_CH, _K), jnp.int32),
            pltpu.VMEM((_CH, _K), jnp.int32),
            pltpu.VMEM((_K, _D), jnp.float32),
            pltpu.VMEM_SHARED((_N, _D), jnp.float32),
            pltpu.SemaphoreType.DMA,
        ],
    )
    return sc_degree, sc_edge_sum


_RB = 2000
_NB = _N // _RB


def _tc_dinv_body(degp_ref, dinv_ref):
    deg = degp_ref[0, :, 0] + degp_ref[1, :, 0] + 1.0
    dinv_ref[...] = lax.rsqrt(deg)[:, None]


_tc_dinv = pl.pallas_call(
    _tc_dinv_body,
    grid=(_N // _RB,),
    in_specs=[pl.BlockSpec((2, _RB, _D), lambda i: (0, i, 0))],
    out_specs=pl.BlockSpec((_RB, 1), lambda i: (i, 0)),
    out_shape=jax.ShapeDtypeStruct((_N, 1), jnp.float32),
)


def _tc_pre_body(dinv_ref, x_ref, w1_ref, wm1_ref, bm1_ref,
                 y1_ref, m_ref):
    dinv = dinv_ref[...]
    xb = x_ref[...]
    z1 = jnp.dot(xb, w1_ref[...], preferred_element_type=jnp.float32,
                 precision=lax.Precision.HIGHEST)
    y1_ref[...] = dinv * z1
    zm = jnp.dot(xb, wm1_ref[...], preferred_element_type=jnp.float32,
                 precision=lax.Precision.HIGHEST)
    m_ref[...] = jnp.maximum(zm + bm1_ref[...], 0.0)


_tc_pre = pl.pallas_call(
    _tc_pre_body,
    grid=(_NB,),
    in_specs=[
        pl.BlockSpec((_RB, 1), lambda i: (i, 0)),
        pl.BlockSpec((_RB, _D), lambda i: (i, 0)),
        pl.BlockSpec((_D, _D), lambda i: (0, 0)),
        pl.BlockSpec((_D, _D), lambda i: (0, 0)),
        pl.BlockSpec((1, _D), lambda i: (0, 0)),
    ],
    out_specs=[
        pl.BlockSpec((_RB, _D), lambda i: (i, 0)),
        pl.BlockSpec((_RB, _D), lambda i: (i, 0)),
    ],
    out_shape=[
        jax.ShapeDtypeStruct((_N, _D), jnp.float32),
        jax.ShapeDtypeStruct((_N, _D), jnp.float32),
    ],
)


def _tc_mid_body(s_ref, y1_ref, dinv_ref, b1_ref, w2_ref, y2_ref):
    dinv = dinv_ref[...]
    full = s_ref[0] + s_ref[1] + y1_ref[...]
    h = jnp.maximum(dinv * full + b1_ref[...], 0.0)
    z2 = jnp.dot(h, w2_ref[...], preferred_element_type=jnp.float32,
                 precision=lax.Precision.HIGHEST)
    y2_ref[...] = dinv * z2


_tc_mid = pl.pallas_call(
    _tc_mid_body,
    grid=(_NB,),
    in_specs=[
        pl.BlockSpec((2, _RB, _D), lambda i: (0, i, 0)),
        pl.BlockSpec((_RB, _D), lambda i: (i, 0)),
        pl.BlockSpec((_RB, 1), lambda i: (i, 0)),
        pl.BlockSpec((1, _D), lambda i: (0, 0)),
        pl.BlockSpec((_D, _D), lambda i: (0, 0)),
    ],
    out_specs=pl.BlockSpec((_RB, _D), lambda i: (i, 0)),
    out_shape=jax.ShapeDtypeStruct((_N, _D), jnp.float32),
)


def _tc_post_body(s_ref, y2_ref, dinv_ref, b2_ref, m_ref, wm2_ref, bm2_ref,
                  gw_ref, mw_ref, out_ref):
    dinv = dinv_ref[...]
    full = s_ref[0] + s_ref[1] + y2_ref[...]
    gcn = dinv * full + b2_ref[...]
    zm = jnp.dot(m_ref[...], wm2_ref[...], preferred_element_type=jnp.float32,
                 precision=lax.Precision.HIGHEST)
    mlp = zm + bm2_ref[...]
    gwv = gw_ref[0, 0]
    mwv = mw_ref[0, 0]
    tot = gwv + mwv
    out_ref[...] = (gwv / tot) * gcn + (mwv / tot) * mlp


_tc_post = pl.pallas_call(
    _tc_post_body,
    grid=(_NB,),
    in_specs=[
        pl.BlockSpec((2, _RB, _D), lambda i: (0, i, 0)),
        pl.BlockSpec((_RB, _D), lambda i: (i, 0)),
        pl.BlockSpec((_RB, 1), lambda i: (i, 0)),
        pl.BlockSpec((1, _D), lambda i: (0, 0)),
        pl.BlockSpec((_RB, _D), lambda i: (i, 0)),
        pl.BlockSpec((_D, _D), lambda i: (0, 0)),
        pl.BlockSpec((1, _D), lambda i: (0, 0)),
        pl.BlockSpec((1, 1), lambda i: (0, 0)),
        pl.BlockSpec((1, 1), lambda i: (0, 0)),
    ],
    out_specs=pl.BlockSpec((_RB, _D), lambda i: (i, 0)),
    out_shape=jax.ShapeDtypeStruct((_N, _D), jnp.float32),
)


@jax.jit
def kernel(x, edge_index, W1, b1, W2, b2, Wm1, bm1, Wm2, bm2,
           gcn_weight, mlp_weight):
    srcs = edge_index[0].reshape(_NW, _CH, _K)
    dsts = edge_index[1].reshape(_NW, _CH, _K)
    zeros2d = jnp.zeros((_N, _D), jnp.float32)
    ones2d = jnp.ones((_K, _D), jnp.float32)
    _sc_degree, _sc_edge_sum = _sc_kernels()

    deg_parts = _sc_degree(dsts, ones2d, zeros2d).reshape(_NC, _N, _D)
    dinv = _tc_dinv(deg_parts)
    y1, m = _tc_pre(dinv, x, W1, Wm1, bm1.reshape(1, _D))

    s1 = _sc_edge_sum(y1, srcs, dsts, zeros2d).reshape(_NC, _N, _D)
    y2 = _tc_mid(s1, y1, dinv, b1.reshape(1, _D), W2)

    s2 = _sc_edge_sum(y2, srcs, dsts, zeros2d).reshape(_NC, _N, _D)
    out = _tc_post(s2, y2, dinv, b2.reshape(1, _D), m, Wm2,
                   bm2.reshape(1, _D),
                   gcn_weight.reshape(1, 1), mlp_weight.reshape(1, 1))
    return out


# speedup vs baseline: 22.0298x; 1.2534x over previous
import functools

import jax
import jax.numpy as jnp
from jax import lax
from jax.experimental import pallas as pl
from jax.experimental.pallas import tpu as pltpu
from jax.experimental.pallas import tpu_sc as plsc

_N = 10000
_E = 320000
_D = 128

_NC = 2
_NS = 16
_NW = _NC * _NS
_EPW = _E // _NW
_K = 100
_CH = _EPW // _K
_NH = 2
_CH2 = _CH // _NH
_W = 4
_ZT = 10
_RPT = _N // _ZT

def _sc_degree_body(dsts_hbm, ones_hbm, zeros_hbm, out_hbm,
                    dst_v, ones_v, acc_sh, ssem):
    cid = lax.axis_index("c")
    sid = lax.axis_index("s")
    wid = cid * _NS + sid
    pltpu.sync_copy(ones_hbm, ones_v)

    @pl.when(sid < _ZT)
    def _():
        pltpu.sync_copy(zeros_hbm.at[pl.ds(sid * _RPT, _RPT)],
                        acc_sh.at[pl.ds(sid * _RPT, _RPT)])

    plsc.subcore_barrier()

    for h in range(_NH):
        pltpu.sync_copy(dsts_hbm.at[wid * _NH + h], dst_v)

        def body(j, carry):
            pltpu.async_copy(ones_v, acc_sh.at[dst_v.at[j]], ssem, add=True)

            @pl.when(j >= _W)
            def _():
                pltpu.make_async_copy(ones_v, acc_sh.at[dst_v.at[j]],
                                      ssem).wait()

            return carry

        lax.fori_loop(0, _CH2, body, 0)
        for _w in range(_W):
            pltpu.make_async_copy(ones_v, acc_sh.at[dst_v.at[0]], ssem).wait()

    plsc.subcore_barrier()

    @pl.when(sid < _ZT)
    def _():
        pltpu.sync_copy(acc_sh.at[pl.ds(sid * _RPT, _RPT)],
                        out_hbm.at[pl.ds(cid * _N + sid * _RPT, _RPT)])


def _sc_edge_sum_body(y_hbm, srcs_hbm, dsts_hbm, zeros_hbm, out_hbm,
                      src_v, dst_v, rows0_v, rows1_v, acc_sh,
                      gsem0, gsem1, ssem0, ssem1):
    cid = lax.axis_index("c")
    sid = lax.axis_index("s")
    wid = cid * _NS + sid
    @pl.when(sid < _ZT)
    def _():
        pltpu.sync_copy(zeros_hbm.at[pl.ds(sid * _RPT, _RPT)],
                        acc_sh.at[pl.ds(sid * _RPT, _RPT)])

    plsc.subcore_barrier()

    for h in range(_NH):
        pltpu.sync_copy(srcs_hbm.at[wid * _NH + h], src_v)
        pltpu.sync_copy(dsts_hbm.at[wid * _NH + h], dst_v)
        pltpu.async_copy(y_hbm.at[src_v.at[0]], rows0_v, gsem0)
        pltpu.async_copy(y_hbm.at[src_v.at[1]], rows1_v, gsem1)

        def body(i, carry):
            j0 = 2 * i
            j1 = j0 + 1
            pltpu.make_async_copy(y_hbm.at[src_v.at[j0]], rows0_v, gsem0).wait()
            pltpu.async_copy(rows0_v, acc_sh.at[dst_v.at[j0]], ssem0, add=True)
            pltpu.make_async_copy(y_hbm.at[src_v.at[j1]], rows1_v, gsem1).wait()
            pltpu.async_copy(rows1_v, acc_sh.at[dst_v.at[j1]], ssem1, add=True)
            pltpu.make_async_copy(rows0_v, acc_sh.at[dst_v.at[j0]], ssem0).wait()

            @pl.when(j0 + 2 < _CH2)
            def _():
                pltpu.async_copy(y_hbm.at[src_v.at[j0 + 2]], rows0_v, gsem0)

            pltpu.make_async_copy(rows1_v, acc_sh.at[dst_v.at[j1]], ssem1).wait()

            @pl.when(j1 + 2 < _CH2)
            def _():
                pltpu.async_copy(y_hbm.at[src_v.at[j1 + 2]], rows1_v, gsem1)

            return carry

        lax.fori_loop(0, _CH2 // 2, body, 0)

    plsc.subcore_barrier()

    @pl.when(sid < _ZT)
    def _():
        pltpu.sync_copy(acc_sh.at[pl.ds(sid * _RPT, _RPT)],
                        out_hbm.at[pl.ds(cid * _N + sid * _RPT, _RPT)])


@functools.lru_cache(maxsize=None)
def _sc_kernels():
    mesh = plsc.VectorSubcoreMesh(core_axis_name="c", subcore_axis_name="s",
                                  num_cores=_NC, num_subcores=_NS)
    sc_degree = pl.kernel(
        _sc_degree_body,
        out_type=jax.ShapeDtypeStruct((_NC * _N, _D), jnp.float32),
        mesh=mesh,
        scratch_types=[
            pltpu.VMEM((_CH2, _K), jnp.int32),
            pltpu.VMEM((_K, _D), jnp.float32),
            pltpu.VMEM_SHARED((_N, _D), jnp.float32),
            pltpu.SemaphoreType.DMA,
        ],
    )
    sc_edge_sum = pl.kernel(
        _sc_edge_sum_body,
        out_type=jax.ShapeDtypeStruct((_NC * _N, _D), jnp.float32),
        mesh=mesh,
        scratch_types=[
            pltpu.VMEM((_CH2, _K), jnp.int32),
            pltpu.VMEM((_CH2, _K), jnp.int32),
            pltpu.VMEM((_K, _D), jnp.float32),
            pltpu.VMEM((_K, _D), jnp.float32),
            pltpu.VMEM_SHARED((_N, _D), jnp.float32),
            pltpu.SemaphoreType.DMA,
            pltpu.SemaphoreType.DMA,
            pltpu.SemaphoreType.DMA,
            pltpu.SemaphoreType.DMA,
        ],
    )
    return sc_degree, sc_edge_sum


_RB = 2000
_NB = _N // _RB


def _tc_dinv_body(degp_ref, dinv_ref):
    deg = degp_ref[0, :, 0] + degp_ref[1, :, 0] + 1.0
    dinv_ref[...] = lax.rsqrt(deg)[:, None]


_tc_dinv = pl.pallas_call(
    _tc_dinv_body,
    grid=(_N // _RB,),
    in_specs=[pl.BlockSpec((2, _RB, _D), lambda i: (0, i, 0))],
    out_specs=pl.BlockSpec((_RB, 1), lambda i: (i, 0)),
    out_shape=jax.ShapeDtypeStruct((_N, 1), jnp.float32),
)


def _tc_pre_body(dinv_ref, x_ref, w1_ref, wm1_ref, bm1_ref,
                 y1_ref, m_ref):
    dinv = dinv_ref[...]
    xb = x_ref[...]
    z1 = jnp.dot(xb, w1_ref[...], preferred_element_type=jnp.float32,
                 precision=lax.Precision.HIGHEST)
    y1_ref[...] = dinv * z1
    zm = jnp.dot(xb, wm1_ref[...], preferred_element_type=jnp.float32,
                 precision=lax.Precision.HIGHEST)
    m_ref[...] = jnp.maximum(zm + bm1_ref[...], 0.0)


_tc_pre = pl.pallas_call(
    _tc_pre_body,
    grid=(_NB,),
    in_specs=[
        pl.BlockSpec((_RB, 1), lambda i: (i, 0)),
        pl.BlockSpec((_RB, _D), lambda i: (i, 0)),
        pl.BlockSpec((_D, _D), lambda i: (0, 0)),
        pl.BlockSpec((_D, _D), lambda i: (0, 0)),
        pl.BlockSpec((1, _D), lambda i: (0, 0)),
    ],
    out_specs=[
        pl.BlockSpec((_RB, _D), lambda i: (i, 0)),
        pl.BlockSpec((_RB, _D), lambda i: (i, 0)),
    ],
    out_shape=[
        jax.ShapeDtypeStruct((_N, _D), jnp.float32),
        jax.ShapeDtypeStruct((_N, _D), jnp.float32),
    ],
)


def _tc_mid_body(s_ref, y1_ref, dinv_ref, b1_ref, w2_ref, y2_ref):
    dinv = dinv_ref[...]
    full = s_ref[0] + s_ref[1] + y1_ref[...]
    h = jnp.maximum(dinv * full + b1_ref[...], 0.0)
    z2 = jnp.dot(h, w2_ref[...], preferred_element_type=jnp.float32,
                 precision=lax.Precision.HIGHEST)
    y2_ref[...] = dinv * z2


_tc_mid = pl.pallas_call(
    _tc_mid_body,
    grid=(_NB,),
    in_specs=[
        pl.BlockSpec((2, _RB, _D), lambda i: (0, i, 0)),
        pl.BlockSpec((_RB, _D), lambda i: (i, 0)),
        pl.BlockSpec((_RB, 1), lambda i: (i, 0)),
        pl.BlockSpec((1, _D), lambda i: (0, 0)),
        pl.BlockSpec((_D, _D), lambda i: (0, 0)),
    ],
    out_specs=pl.BlockSpec((_RB, _D), lambda i: (i, 0)),
    out_shape=jax.ShapeDtypeStruct((_N, _D), jnp.float32),
)


def _tc_post_body(s_ref, y2_ref, dinv_ref, b2_ref, m_ref, wm2_ref, bm2_ref,
                  gw_ref, mw_ref, out_ref):
    dinv = dinv_ref[...]
    full = s_ref[0] + s_ref[1] + y2_ref[...]
    gcn = dinv * full + b2_ref[...]
    zm = jnp.dot(m_ref[...], wm2_ref[...], preferred_element_type=jnp.float32,
                 precision=lax.Precision.HIGHEST)
    mlp = zm + bm2_ref[...]
    gwv = gw_ref[0, 0]
    mwv = mw_ref[0, 0]
    tot = gwv + mwv
    out_ref[...] = (gwv / tot) * gcn + (mwv / tot) * mlp


_tc_post = pl.pallas_call(
    _tc_post_body,
    grid=(_NB,),
    in_specs=[
        pl.BlockSpec((2, _RB, _D), lambda i: (0, i, 0)),
        pl.BlockSpec((_RB, _D), lambda i: (i, 0)),
        pl.BlockSpec((_RB, 1), lambda i: (i, 0)),
        pl.BlockSpec((1, _D), lambda i: (0, 0)),
        pl.BlockSpec((_RB, _D), lambda i: (i, 0)),
        pl.BlockSpec((_D, _D), lambda i: (0, 0)),
        pl.BlockSpec((1, _D), lambda i: (0, 0)),
        pl.BlockSpec((1, 1), lambda i: (0, 0)),
        pl.BlockSpec((1, 1), lambda i: (0, 0)),
    ],
    out_specs=pl.BlockSpec((_RB, _D), lambda i: (i, 0)),
    out_shape=jax.ShapeDtypeStruct((_N, _D), jnp.float32),
)


@jax.jit
def kernel(x, edge_index, W1, b1, W2, b2, Wm1, bm1, Wm2, bm2,
           gcn_weight, mlp_weight):
    srcs = edge_index[0].reshape(_NW * _NH, _CH2, _K)
    dsts = edge_index[1].reshape(_NW * _NH, _CH2, _K)
    zeros2d = jnp.zeros((_N, _D), jnp.float32)
    ones2d = jnp.ones((_K, _D), jnp.float32)
    _sc_degree, _sc_edge_sum = _sc_kernels()

    deg_parts = _sc_degree(dsts, ones2d, zeros2d).reshape(_NC, _N, _D)
    dinv = _tc_dinv(deg_parts)
    y1, m = _tc_pre(dinv, x, W1, Wm1, bm1.reshape(1, _D))

    s1 = _sc_edge_sum(y1, srcs, dsts, zeros2d).reshape(_NC, _N, _D)
    y2 = _tc_mid(s1, y1, dinv, b1.reshape(1, _D), W2)

    s2 = _sc_edge_sum(y2, srcs, dsts, zeros2d).reshape(_NC, _N, _D)
    out = _tc_post(s2, y2, dinv, b2.reshape(1, _D), m, Wm2,
                   bm2.reshape(1, _D),
                   gcn_weight.reshape(1, 1), mlp_weight.reshape(1, 1))
    return out


# speedup vs baseline: 22.4618x; 1.0196x over previous
import functools

import jax
import jax.numpy as jnp
from jax import lax
from jax.experimental import pallas as pl
from jax.experimental.pallas import tpu as pltpu
from jax.experimental.pallas import tpu_sc as plsc

_N = 10000
_E = 320000
_D = 128

_NC = 2
_NS = 16
_NW = _NC * _NS
_EPW = _E // _NW
_K = 100
_CH = _EPW // _K
_NH = 2
_CH2 = _CH // _NH
_W = 4
_ZT = 10
_RPT = _N // _ZT

def _sc_degree_body(dsts_hbm, ones_hbm, zeros_hbm, out_hbm,
                    dst_v, ones_v, acc_sh, ssem):
    cid = lax.axis_index("c")
    sid = lax.axis_index("s")
    wid = cid * _NS + sid
    pltpu.sync_copy(ones_hbm, ones_v)

    @pl.when(sid < _ZT)
    def _():
        pltpu.sync_copy(zeros_hbm.at[pl.ds(sid * _RPT, _RPT)],
                        acc_sh.at[pl.ds(sid * _RPT, _RPT)])

    plsc.subcore_barrier()

    for h in range(_NH):
        pltpu.sync_copy(dsts_hbm.at[wid * _NH + h], dst_v)

        def body(j, carry):
            pltpu.async_copy(ones_v, acc_sh.at[dst_v.at[j]], ssem, add=True)

            @pl.when(j >= _W)
            def _():
                pltpu.make_async_copy(ones_v, acc_sh.at[dst_v.at[j]],
                                      ssem).wait()

            return carry

        lax.fori_loop(0, _CH2, body, 0)
        for _w in range(_W):
            pltpu.make_async_copy(ones_v, acc_sh.at[dst_v.at[0]], ssem).wait()

    plsc.subcore_barrier()

    @pl.when(sid < _ZT)
    def _():
        pltpu.sync_copy(acc_sh.at[pl.ds(sid * _RPT, _RPT)],
                        out_hbm.at[pl.ds(cid * _N + sid * _RPT, _RPT)])


def _sc_edge_sum_body(y_hbm, srcs_hbm, dsts_hbm, zeros_hbm, out_hbm,
                      src_v, dst_v, rows0_v, rows1_v, acc_sh,
                      gsem0, gsem1, ssem0, ssem1):
    cid = lax.axis_index("c")
    sid = lax.axis_index("s")
    wid = cid * _NS + sid
    @pl.when(sid < _ZT)
    def _():
        pltpu.sync_copy(zeros_hbm.at[pl.ds(sid * _RPT, _RPT)],
                        acc_sh.at[pl.ds(sid * _RPT, _RPT)])

    plsc.subcore_barrier()

    for h in range(_NH):
        pltpu.sync_copy(srcs_hbm.at[wid * _NH + h], src_v)
        pltpu.sync_copy(dsts_hbm.at[wid * _NH + h], dst_v)
        pltpu.async_copy(y_hbm.at[src_v.at[0]], rows0_v, gsem0)
        pltpu.async_copy(y_hbm.at[src_v.at[1]], rows1_v, gsem1)

        def body(i, carry):
            j0 = 2 * i
            j1 = j0 + 1
            pltpu.make_async_copy(y_hbm.at[src_v.at[j0]], rows0_v, gsem0).wait()
            pltpu.async_copy(rows0_v, acc_sh.at[dst_v.at[j0]], ssem0, add=True)
            pltpu.make_async_copy(y_hbm.at[src_v.at[j1]], rows1_v, gsem1).wait()
            pltpu.async_copy(rows1_v, acc_sh.at[dst_v.at[j1]], ssem1, add=True)
            pltpu.make_async_copy(rows0_v, acc_sh.at[dst_v.at[j0]], ssem0).wait()

            @pl.when(j0 + 2 < _CH2)
            def _():
                pltpu.async_copy(y_hbm.at[src_v.at[j0 + 2]], rows0_v, gsem0)

            pltpu.make_async_copy(rows1_v, acc_sh.at[dst_v.at[j1]], ssem1).wait()

            @pl.when(j1 + 2 < _CH2)
            def _():
                pltpu.async_copy(y_hbm.at[src_v.at[j1 + 2]], rows1_v, gsem1)

            return carry

        lax.fori_loop(0, _CH2 // 2, body, 0)

    plsc.subcore_barrier()

    @pl.when(sid < _ZT)
    def _():
        pltpu.sync_copy(acc_sh.at[pl.ds(sid * _RPT, _RPT)],
                        out_hbm.at[pl.ds(cid * _N + sid * _RPT, _RPT)])


@functools.lru_cache(maxsize=None)
def _sc_kernels():
    mesh = plsc.VectorSubcoreMesh(core_axis_name="c", subcore_axis_name="s",
                                  num_cores=_NC, num_subcores=_NS)
    sc_degree = pl.kernel(
        _sc_degree_body,
        out_type=jax.ShapeDtypeStruct((_NC * _N, _D), jnp.float32),
        mesh=mesh,
        scratch_types=[
            pltpu.VMEM((_CH2, _K), jnp.int32),
            pltpu.VMEM((_K, _D), jnp.float32),
            pltpu.VMEM_SHARED((_N, _D), jnp.float32),
            pltpu.SemaphoreType.DMA,
        ],
    )
    sc_edge_sum = pl.kernel(
        _sc_edge_sum_body,
        out_type=jax.ShapeDtypeStruct((_NC * _N, _D), jnp.float32),
        mesh=mesh,
        scratch_types=[
            pltpu.VMEM((_CH2, _K), jnp.int32),
            pltpu.VMEM((_CH2, _K), jnp.int32),
            pltpu.VMEM((_K, _D), jnp.float32),
            pltpu.VMEM((_K, _D), jnp.float32),
            pltpu.VMEM_SHARED((_N, _D), jnp.float32),
            pltpu.SemaphoreType.DMA,
            pltpu.SemaphoreType.DMA,
            pltpu.SemaphoreType.DMA,
            pltpu.SemaphoreType.DMA,
        ],
    )
    return sc_degree, sc_edge_sum


_RB = 2000
_NB = _N // _RB


def _tc_dinv_body(degp_ref, z1_ref, dinv_ref, y1_ref):
    deg = degp_ref[0, :, 0] + degp_ref[1, :, 0] + 1.0
    dinv = lax.rsqrt(deg)[:, None]
    dinv_ref[...] = dinv
    y1_ref[...] = dinv * z1_ref[...]


_tc_dinv = pl.pallas_call(
    _tc_dinv_body,
    grid=(_NB,),
    in_specs=[
        pl.BlockSpec((2, _RB, _D), lambda i: (0, i, 0)),
        pl.BlockSpec((_RB, _D), lambda i: (i, 0)),
    ],
    out_specs=[
        pl.BlockSpec((_RB, 1), lambda i: (i, 0)),
        pl.BlockSpec((_RB, _D), lambda i: (i, 0)),
    ],
    out_shape=[
        jax.ShapeDtypeStruct((_N, 1), jnp.float32),
        jax.ShapeDtypeStruct((_N, _D), jnp.float32),
    ],
)


def _tc_mats_body(x_ref, w1_ref, wm1_ref, bm1_ref, z1_ref, m_ref):
    xb = x_ref[...]
    z1_ref[...] = jnp.dot(xb, w1_ref[...], preferred_element_type=jnp.float32,
                          precision=lax.Precision.HIGHEST)
    zm = jnp.dot(xb, wm1_ref[...], preferred_element_type=jnp.float32,
                 precision=lax.Precision.HIGHEST)
    m_ref[...] = jnp.maximum(zm + bm1_ref[...], 0.0)


_tc_mats = pl.pallas_call(
    _tc_mats_body,
    grid=(_NB,),
    in_specs=[
        pl.BlockSpec((_RB, _D), lambda i: (i, 0)),
        pl.BlockSpec((_D, _D), lambda i: (0, 0)),
        pl.BlockSpec((_D, _D), lambda i: (0, 0)),
        pl.BlockSpec((1, _D), lambda i: (0, 0)),
    ],
    out_specs=[
        pl.BlockSpec((_RB, _D), lambda i: (i, 0)),
        pl.BlockSpec((_RB, _D), lambda i: (i, 0)),
    ],
    out_shape=[
        jax.ShapeDtypeStruct((_N, _D), jnp.float32),
        jax.ShapeDtypeStruct((_N, _D), jnp.float32),
    ],
)


def _tc_mlp2_body(m_ref, wm2_ref, bm2_ref, mlp_ref):
    mlp_ref[...] = jnp.dot(m_ref[...], wm2_ref[...],
                           preferred_element_type=jnp.float32,
                           precision=lax.Precision.HIGHEST) + bm2_ref[...]


_tc_mlp2 = pl.pallas_call(
    _tc_mlp2_body,
    grid=(_NB,),
    in_specs=[
        pl.BlockSpec((_RB, _D), lambda i: (i, 0)),
        pl.BlockSpec((_D, _D), lambda i: (0, 0)),
        pl.BlockSpec((1, _D), lambda i: (0, 0)),
    ],
    out_specs=pl.BlockSpec((_RB, _D), lambda i: (i, 0)),
    out_shape=jax.ShapeDtypeStruct((_N, _D), jnp.float32),
)


def _tc_mid_body(s_ref, y1_ref, dinv_ref, b1_ref, w2_ref, y2_ref):
    dinv = dinv_ref[...]
    full = s_ref[0] + s_ref[1] + y1_ref[...]
    h = jnp.maximum(dinv * full + b1_ref[...], 0.0)
    z2 = jnp.dot(h, w2_ref[...], preferred_element_type=jnp.float32,
                 precision=lax.Precision.HIGHEST)
    y2_ref[...] = dinv * z2


_tc_mid = pl.pallas_call(
    _tc_mid_body,
    grid=(_NB,),
    in_specs=[
        pl.BlockSpec((2, _RB, _D), lambda i: (0, i, 0)),
        pl.BlockSpec((_RB, _D), lambda i: (i, 0)),
        pl.BlockSpec((_RB, 1), lambda i: (i, 0)),
        pl.BlockSpec((1, _D), lambda i: (0, 0)),
        pl.BlockSpec((_D, _D), lambda i: (0, 0)),
    ],
    out_specs=pl.BlockSpec((_RB, _D), lambda i: (i, 0)),
    out_shape=jax.ShapeDtypeStruct((_N, _D), jnp.float32),
)


def _tc_post_body(s_ref, y2_ref, dinv_ref, b2_ref, mlp_ref,
                  gw_ref, mw_ref, out_ref):
    dinv = dinv_ref[...]
    full = s_ref[0] + s_ref[1] + y2_ref[...]
    gcn = dinv * full + b2_ref[...]
    mlp = mlp_ref[...]
    gwv = gw_ref[0, 0]
    mwv = mw_ref[0, 0]
    tot = gwv + mwv
    out_ref[...] = (gwv / tot) * gcn + (mwv / tot) * mlp


_tc_post = pl.pallas_call(
    _tc_post_body,
    grid=(_NB,),
    in_specs=[
        pl.BlockSpec((2, _RB, _D), lambda i: (0, i, 0)),
        pl.BlockSpec((_RB, _D), lambda i: (i, 0)),
        pl.BlockSpec((_RB, 1), lambda i: (i, 0)),
        pl.BlockSpec((1, _D), lambda i: (0, 0)),
        pl.BlockSpec((_RB, _D), lambda i: (i, 0)),
        pl.BlockSpec((1, 1), lambda i: (0, 0)),
        pl.BlockSpec((1, 1), lambda i: (0, 0)),
    ],
    out_specs=pl.BlockSpec((_RB, _D), lambda i: (i, 0)),
    out_shape=jax.ShapeDtypeStruct((_N, _D), jnp.float32),
)


@jax.jit
def kernel(x, edge_index, W1, b1, W2, b2, Wm1, bm1, Wm2, bm2,
           gcn_weight, mlp_weight):
    srcs = edge_index[0].reshape(_NW * _NH, _CH2, _K)
    dsts = edge_index[1].reshape(_NW * _NH, _CH2, _K)
    zeros2d = jnp.zeros((_N, _D), jnp.float32)
    ones2d = jnp.ones((_K, _D), jnp.float32)
    _sc_degree, _sc_edge_sum = _sc_kernels()

    deg_parts = _sc_degree(dsts, ones2d, zeros2d).reshape(_NC, _N, _D)
    z1, m = _tc_mats(x, W1, Wm1, bm1.reshape(1, _D))
    dinv, y1 = _tc_dinv(deg_parts, z1)

    s1 = _sc_edge_sum(y1, srcs, dsts, zeros2d).reshape(_NC, _N, _D)
    mlp_out = _tc_mlp2(m, Wm2, bm2.reshape(1, _D))
    y2 = _tc_mid(s1, y1, dinv, b1.reshape(1, _D), W2)

    s2 = _sc_edge_sum(y2, srcs, dsts, zeros2d).reshape(_NC, _N, _D)
    out = _tc_post(s2, y2, dinv, b2.reshape(1, _D), mlp_out,
                   gcn_weight.reshape(1, 1), mlp_weight.reshape(1, 1))
    return out


# speedup vs baseline: 25.4019x; 1.1309x over previous
import functools

import jax
import jax.numpy as jnp
from jax import lax
from jax.experimental import pallas as pl
from jax.experimental.pallas import tpu as pltpu
from jax.experimental.pallas import tpu_sc as plsc

_N = 10000
_E = 320000
_D = 128

_NC = 2
_NS = 16
_NW = _NC * _NS
_EPW = _E // _NW
_K = 100
_CH = _EPW // _K
_NH = 2
_CH2 = _CH // _NH
_KE = 50
_CE = _EPW // _KE
_EH = 4
_CE4 = _CE // _EH
_PD = 4
_EPI = _CE4 % _PD
_W = 4
_ZT = 10
_RPT = _N // _ZT

def _sc_degree_body(dsts_hbm, ones_hbm, zeros_hbm, out_hbm,
                    dst_v, ones_v, acc_sh, ssem):
    cid = lax.axis_index("c")
    sid = lax.axis_index("s")
    wid = cid * _NS + sid
    pltpu.sync_copy(ones_hbm, ones_v)

    @pl.when(sid < _ZT)
    def _():
        pltpu.sync_copy(zeros_hbm.at[pl.ds(sid * _RPT, _RPT)],
                        acc_sh.at[pl.ds(sid * _RPT, _RPT)])

    plsc.subcore_barrier()

    for h in range(_NH):
        pltpu.sync_copy(dsts_hbm.at[wid * _NH + h], dst_v)

        def body(j, carry):
            pltpu.async_copy(ones_v, acc_sh.at[dst_v.at[j]], ssem, add=True)

            @pl.when(j >= _W)
            def _():
                pltpu.make_async_copy(ones_v, acc_sh.at[dst_v.at[j]],
                                      ssem).wait()

            return carry

        lax.fori_loop(0, _CH2, body, 0)
        for _w in range(_W):
            pltpu.make_async_copy(ones_v, acc_sh.at[dst_v.at[0]], ssem).wait()

    plsc.subcore_barrier()

    @pl.when(sid < _ZT)
    def _():
        pltpu.sync_copy(acc_sh.at[pl.ds(sid * _RPT, _RPT)],
                        out_hbm.at[pl.ds(cid * _N + sid * _RPT, _RPT)])


def _sc_edge_sum_body(y_hbm, srcs_hbm, dsts_hbm, zeros_hbm, out_hbm,
                      src_v, dst_v, rows_v, acc_sh, gsems, ssems):
    cid = lax.axis_index("c")
    sid = lax.axis_index("s")
    wid = cid * _NS + sid
    @pl.when(sid < _ZT)
    def _():
        pltpu.sync_copy(zeros_hbm.at[pl.ds(sid * _RPT, _RPT)],
                        acc_sh.at[pl.ds(sid * _RPT, _RPT)])

    plsc.subcore_barrier()

    for h in range(_EH):
        pltpu.sync_copy(srcs_hbm.at[wid * _EH + h], src_v)
        pltpu.sync_copy(dsts_hbm.at[wid * _EH + h], dst_v)
        for b in range(_PD):
            pltpu.async_copy(y_hbm.at[src_v.at[b]], rows_v.at[b], gsems.at[b])

        def body(i, carry):
            j = _PD * i
            for b in range(_PD):
                pltpu.make_async_copy(y_hbm.at[src_v.at[j + b]],
                                      rows_v.at[b], gsems.at[b]).wait()
                pltpu.async_copy(rows_v.at[b], acc_sh.at[dst_v.at[j + b]],
                                 ssems.at[b], add=True)
            for b in range(_PD):
                pltpu.make_async_copy(rows_v.at[b],
                                      acc_sh.at[dst_v.at[j + b]],
                                      ssems.at[b]).wait()

                @pl.when(j + b + _PD < _CE4)
                def _():
                    pltpu.async_copy(y_hbm.at[src_v.at[j + b + _PD]],
                                     rows_v.at[b], gsems.at[b])

            return carry

        lax.fori_loop(0, _CE4 // _PD, body, 0)
        for e in range(_EPI):
            j = _CE4 - _EPI + e
            b = j % _PD
            pltpu.make_async_copy(y_hbm.at[src_v.at[j]],
                                  rows_v.at[b], gsems.at[b]).wait()
            pltpu.async_copy(rows_v.at[b], acc_sh.at[dst_v.at[j]],
                             ssems.at[b], add=True)
        for e in range(_EPI):
            j = _CE4 - _EPI + e
            b = j % _PD
            pltpu.make_async_copy(rows_v.at[b], acc_sh.at[dst_v.at[j]],
                                  ssems.at[b]).wait()

    plsc.subcore_barrier()

    @pl.when(sid < _ZT)
    def _():
        pltpu.sync_copy(acc_sh.at[pl.ds(sid * _RPT, _RPT)],
                        out_hbm.at[pl.ds(cid * _N + sid * _RPT, _RPT)])


@functools.lru_cache(maxsize=None)
def _sc_kernels():
    mesh = plsc.VectorSubcoreMesh(core_axis_name="c", subcore_axis_name="s",
                                  num_cores=_NC, num_subcores=_NS)
    sc_degree = pl.kernel(
        _sc_degree_body,
        out_type=jax.ShapeDtypeStruct((_NC * _N, _D), jnp.float32),
        mesh=mesh,
        scratch_types=[
            pltpu.VMEM((_CH2, _K), jnp.int32),
            pltpu.VMEM((_K, _D), jnp.float32),
            pltpu.VMEM_SHARED((_N, _D), jnp.float32),
            pltpu.SemaphoreType.DMA,
        ],
    )
    sc_edge_sum = pl.kernel(
        _sc_edge_sum_body,
        out_type=jax.ShapeDtypeStruct((_NC * _N, _D), jnp.float32),
        mesh=mesh,
        scratch_types=[
            pltpu.VMEM((_CE4, _KE), jnp.int32),
            pltpu.VMEM((_CE4, _KE), jnp.int32),
            pltpu.VMEM((_PD, _KE, _D), jnp.float32),
            pltpu.VMEM_SHARED((_N, _D), jnp.float32),
            pltpu.SemaphoreType.DMA((_PD,)),
            pltpu.SemaphoreType.DMA((_PD,)),
        ],
    )
    return sc_degree, sc_edge_sum


_RB = 2000
_NB = _N // _RB


def _tc_dinv_body(degp_ref, z1_ref, dinv_ref, y1_ref):
    deg = degp_ref[0, :, 0] + degp_ref[1, :, 0] + 1.0
    dinv = lax.rsqrt(deg)[:, None]
    dinv_ref[...] = dinv
    y1_ref[...] = dinv * z1_ref[...]


_tc_dinv = pl.pallas_call(
    _tc_dinv_body,
    grid=(_NB,),
    in_specs=[
        pl.BlockSpec((2, _RB, _D), lambda i: (0, i, 0)),
        pl.BlockSpec((_RB, _D), lambda i: (i, 0)),
    ],
    out_specs=[
        pl.BlockSpec((_RB, 1), lambda i: (i, 0)),
        pl.BlockSpec((_RB, _D), lambda i: (i, 0)),
    ],
    out_shape=[
        jax.ShapeDtypeStruct((_N, 1), jnp.float32),
        jax.ShapeDtypeStruct((_N, _D), jnp.float32),
    ],
)


def _tc_mats_body(x_ref, w1_ref, wm1_ref, bm1_ref, z1_ref, m_ref):
    xb = x_ref[...]
    z1_ref[...] = jnp.dot(xb, w1_ref[...], preferred_element_type=jnp.float32,
                          precision=lax.Precision.HIGHEST)
    zm = jnp.dot(xb, wm1_ref[...], preferred_element_type=jnp.float32,
                 precision=lax.Precision.HIGHEST)
    m_ref[...] = jnp.maximum(zm + bm1_ref[...], 0.0)


_tc_mats = pl.pallas_call(
    _tc_mats_body,
    grid=(_NB,),
    in_specs=[
        pl.BlockSpec((_RB, _D), lambda i: (i, 0)),
        pl.BlockSpec((_D, _D), lambda i: (0, 0)),
        pl.BlockSpec((_D, _D), lambda i: (0, 0)),
        pl.BlockSpec((1, _D), lambda i: (0, 0)),
    ],
    out_specs=[
        pl.BlockSpec((_RB, _D), lambda i: (i, 0)),
        pl.BlockSpec((_RB, _D), lambda i: (i, 0)),
    ],
    out_shape=[
        jax.ShapeDtypeStruct((_N, _D), jnp.float32),
        jax.ShapeDtypeStruct((_N, _D), jnp.float32),
    ],
)


def _tc_mlp2_body(m_ref, wm2_ref, bm2_ref, mlp_ref):
    mlp_ref[...] = jnp.dot(m_ref[...], wm2_ref[...],
                           preferred_element_type=jnp.float32,
                           precision=lax.Precision.HIGHEST) + bm2_ref[...]


_tc_mlp2 = pl.pallas_call(
    _tc_mlp2_body,
    grid=(_NB,),
    in_specs=[
        pl.BlockSpec((_RB, _D), lambda i: (i, 0)),
        pl.BlockSpec((_D, _D), lambda i: (0, 0)),
        pl.BlockSpec((1, _D), lambda i: (0, 0)),
    ],
    out_specs=pl.BlockSpec((_RB, _D), lambda i: (i, 0)),
    out_shape=jax.ShapeDtypeStruct((_N, _D), jnp.float32),
)


def _tc_mid_body(s_ref, y1_ref, dinv_ref, b1_ref, w2_ref, y2_ref):
    dinv = dinv_ref[...]
    full = s_ref[0] + s_ref[1] + y1_ref[...]
    h = jnp.maximum(dinv * full + b1_ref[...], 0.0)
    z2 = jnp.dot(h, w2_ref[...], preferred_element_type=jnp.float32,
                 precision=lax.Precision.HIGHEST)
    y2_ref[...] = dinv * z2


_tc_mid = pl.pallas_call(
    _tc_mid_body,
    grid=(_NB,),
    in_specs=[
        pl.BlockSpec((2, _RB, _D), lambda i: (0, i, 0)),
        pl.BlockSpec((_RB, _D), lambda i: (i, 0)),
        pl.BlockSpec((_RB, 1), lambda i: (i, 0)),
        pl.BlockSpec((1, _D), lambda i: (0, 0)),
        pl.BlockSpec((_D, _D), lambda i: (0, 0)),
    ],
    out_specs=pl.BlockSpec((_RB, _D), lambda i: (i, 0)),
    out_shape=jax.ShapeDtypeStruct((_N, _D), jnp.float32),
)


def _tc_post_body(s_ref, y2_ref, dinv_ref, b2_ref, mlp_ref,
                  gw_ref, mw_ref, out_ref):
    dinv = dinv_ref[...]
    full = s_ref[0] + s_ref[1] + y2_ref[...]
    gcn = dinv * full + b2_ref[...]
    mlp = mlp_ref[...]
    gwv = gw_ref[0, 0]
    mwv = mw_ref[0, 0]
    tot = gwv + mwv
    out_ref[...] = (gwv / tot) * gcn + (mwv / tot) * mlp


_tc_post = pl.pallas_call(
    _tc_post_body,
    grid=(_NB,),
    in_specs=[
        pl.BlockSpec((2, _RB, _D), lambda i: (0, i, 0)),
        pl.BlockSpec((_RB, _D), lambda i: (i, 0)),
        pl.BlockSpec((_RB, 1), lambda i: (i, 0)),
        pl.BlockSpec((1, _D), lambda i: (0, 0)),
        pl.BlockSpec((_RB, _D), lambda i: (i, 0)),
        pl.BlockSpec((1, 1), lambda i: (0, 0)),
        pl.BlockSpec((1, 1), lambda i: (0, 0)),
    ],
    out_specs=pl.BlockSpec((_RB, _D), lambda i: (i, 0)),
    out_shape=jax.ShapeDtypeStruct((_N, _D), jnp.float32),
)


@jax.jit
def kernel(x, edge_index, W1, b1, W2, b2, Wm1, bm1, Wm2, bm2,
           gcn_weight, mlp_weight):
    srcs = edge_index[0].reshape(_NW * _EH, _CE4, _KE)
    dsts = edge_index[1].reshape(_NW * _EH, _CE4, _KE)
    dsts_deg = edge_index[1].reshape(_NW * _NH, _CH2, _K)
    zeros2d = jnp.zeros((_N, _D), jnp.float32)
    ones2d = jnp.ones((_K, _D), jnp.float32)
    _sc_degree, _sc_edge_sum = _sc_kernels()

    deg_parts = _sc_degree(dsts_deg, ones2d, zeros2d).reshape(_NC, _N, _D)
    z1, m = _tc_mats(x, W1, Wm1, bm1.reshape(1, _D))
    dinv, y1 = _tc_dinv(deg_parts, z1)

    s1 = _sc_edge_sum(y1, srcs, dsts, zeros2d).reshape(_NC, _N, _D)
    mlp_out = _tc_mlp2(m, Wm2, bm2.reshape(1, _D))
    y2 = _tc_mid(s1, y1, dinv, b1.reshape(1, _D), W2)

    s2 = _sc_edge_sum(y2, srcs, dsts, zeros2d).reshape(_NC, _N, _D)
    out = _tc_post(s2, y2, dinv, b2.reshape(1, _D), mlp_out,
                   gcn_weight.reshape(1, 1), mlp_weight.reshape(1, 1))
    return out
